# Initial kernel scaffold; baseline (speedup 1.0000x reference)
import jax
import jax.numpy as jnp
from jax.experimental import pallas as pl


def kernel(x, z, atomref_weight):
    raise NotImplementedError("write your pallas kernel here")


# speedup vs baseline: 76.8104x; 76.8104x over previous
import functools

import jax
import jax.numpy as jnp
from jax import lax
from jax.experimental import pallas as pl
from jax.experimental.pallas import tpu as pltpu
from jax.experimental.pallas import tpu_sc as plsc

_N = 1_000_000
_TABLE = 128
_LANES = 16

_info = plsc.get_sparse_core_info()
_NC = _info.num_cores
_NS = _info.num_subcores
_NW = _NC * _NS

_PER_W = -(-_N // _NW)
_PER_W += (-_PER_W) % _LANES
_NPAD = _PER_W * _NW
_NVEC = _PER_W // _LANES


@functools.partial(
    pl.kernel,
    out_type=jax.ShapeDtypeStruct((_NPAD,), jnp.float32),
    mesh=plsc.VectorSubcoreMesh(core_axis_name="c", subcore_axis_name="s"),
    compiler_params=pltpu.CompilerParams(needs_layout_passes=False),
    scratch_types=[
        pltpu.VMEM((_TABLE,), jnp.float32),
        pltpu.VMEM((_PER_W,), jnp.int32),
        pltpu.VMEM((_PER_W,), jnp.float32),
        pltpu.VMEM((_PER_W,), jnp.float32),
    ],
)
def _atomref_sc(x_hbm, z_hbm, tab_hbm, out_hbm, tab_v, z_v, x_v, o_v):
    wid = lax.axis_index("s") * _NC + lax.axis_index("c")
    base = wid * _PER_W

    pltpu.sync_copy(tab_hbm, tab_v)
    pltpu.sync_copy(z_hbm.at[pl.ds(base, _PER_W)], z_v)
    pltpu.sync_copy(x_hbm.at[pl.ds(base, _PER_W)], x_v)

    def body(i, _):
        off = i * _LANES
        zv = z_v[pl.ds(off, _LANES)]
        xv = x_v[pl.ds(off, _LANES)]
        gv = plsc.load_gather(tab_v, [zv])
        o_v[pl.ds(off, _LANES)] = xv + gv
        return 0

    lax.fori_loop(0, _NVEC, body, 0)

    pltpu.sync_copy(o_v, out_hbm.at[pl.ds(base, _PER_W)])


def kernel(x, z, atomref_weight):
    xf = jnp.ravel(x).astype(jnp.float32)
    zi = jnp.ravel(z).astype(jnp.int32)
    tab = jnp.pad(jnp.ravel(atomref_weight).astype(jnp.float32),
                  (0, _TABLE - atomref_weight.shape[0]))
    xp = jnp.pad(xf, (0, _NPAD - _N))
    zp = jnp.pad(zi, (0, _NPAD - _N))
    out = _atomref_sc(xp, zp, tab)
    return out[:_N].reshape(_N, 1)
